# baseline (speedup 1.0000x reference)
import functools

import jax
import jax.numpy as jnp
from jax import lax
from jax.experimental import pallas as pl
from jax.experimental.pallas import tpu as pltpu
from jax.experimental.pallas import tpu_sc as plsc

D = 32
B = 16384
V = 1000000

S_TC = 4096
B_SC = B - S_TC

_info = plsc.get_sparse_core_info()
_NC, _NS = _info.num_cores, _info.num_subcores
NW = _NC * _NS
BPW = B_SC // NW
GS = 16
NBLK = BPW // 128

_mesh = plsc.VectorSubcoreMesh(core_axis_name="c", subcore_axis_name="s")


@functools.partial(
    pl.kernel,
    mesh=_mesh,
    out_type=jax.ShapeDtypeStruct((D, B_SC), jnp.float32),
    scratch_types=[
        pltpu.VMEM((BPW,), jnp.int32),
        pltpu.VMEM((GS, D, 128), jnp.float32),
        pltpu.VMEM((D, 128), jnp.float32),
        pltpu.SemaphoreType.DMA,
        pltpu.SemaphoreType.DMA,
    ],
    compiler_params=pltpu.CompilerParams(needs_layout_passes=False),
)
def _sc_gather(idx_hbm, embT_hbm, outT_hbm, idx_v, blk_v, ob_v, gsem, osem):
    wid = lax.axis_index("s") * _NC + lax.axis_index("c")
    base = wid * BPW
    pltpu.sync_copy(idx_hbm.at[pl.ds(base, BPW)], idx_v)
    iota = lax.iota(jnp.int32, 16)

    def block(blki, carry):
        bb = blki * 128
        for sub in range(128 // GS):
            rv = idx_v[pl.ds(bb + sub * GS, GS)]
            copies = []
            lanes = []
            for i in range(GS):
                r = rv[i]
                w0 = pl.multiple_of(
                    lax.shift_left(lax.shift_right_logical(r, 7), 7), 128
                )
                lanes.append(r - w0)
                copies.append(
                    pltpu.async_copy(
                        embT_hbm.at[:, pl.ds(w0, 128)], blk_v.at[i], gsem
                    )
                )
            for c in copies:
                c.wait()
            for i in range(GS):
                lane = jnp.full((16,), lanes[i], jnp.int32)
                row = jnp.full((16,), i, jnp.int32)
                col = jnp.full((16,), sub * GS + i, jnp.int32)
                lo = plsc.load_gather(blk_v, [row, iota, lane])
                hi = plsc.load_gather(blk_v, [row, iota + 16, lane])
                plsc.store_scatter(ob_v, [iota, col], lo)
                plsc.store_scatter(ob_v, [iota + 16, col], hi)
        pltpu.async_copy(
            ob_v, outT_hbm.at[:, pl.ds(base + bb, 128)], osem
        ).wait()
        return carry

    lax.fori_loop(0, NBLK, block, 0)


EPG = 8


def _tc_gather_body(wins_ref, lanes_ref, *refs):
    tables = refs[:EPG]
    out_ref = refs[EPG]
    i = pl.program_id(0)
    lane_iota = lax.broadcasted_iota(jnp.int32, (D, 128), 1)
    for k in range(EPG):
        j = lanes_ref[i * EPG + k]
        blk = tables[k][...]
        col = jnp.sum(jnp.where(lane_iota == j, blk, 0.0), axis=1)
        out_ref[k, :] = col


def _tc_in_spec(k):
    return pl.BlockSpec((D, 128), lambda i, wins, lanes, k=k: (0, wins[i * EPG + k]))


_tc_gather = pl.pallas_call(
    _tc_gather_body,
    grid_spec=pltpu.PrefetchScalarGridSpec(
        num_scalar_prefetch=2,
        grid=(S_TC // EPG,),
        in_specs=[_tc_in_spec(k) for k in range(EPG)],
        out_specs=pl.BlockSpec((EPG, D), lambda i, wins, lanes: (i, 0)),
    ),
    out_shape=jax.ShapeDtypeStruct((S_TC, D), jnp.float32),
)


def kernel(node_idx, emb):
    idx = node_idx.astype(jnp.int32)
    embT = emb.T
    idx_tc = idx[:S_TC]
    wins = lax.shift_right_logical(idx_tc, 7)
    lanes = jnp.bitwise_and(idx_tc, 127)
    tc_out = _tc_gather(wins, lanes, *([embT] * EPG))
    sc_outT = _sc_gather(idx[S_TC:], embT)
    return jnp.concatenate([tc_out, sc_outT.T], axis=0)


# speedup vs baseline: 1.6248x; 1.6248x over previous
import functools

import jax
import jax.numpy as jnp
from jax import lax
from jax.experimental import pallas as pl
from jax.experimental.pallas import tpu as pltpu
from jax.experimental.pallas import tpu_sc as plsc

D = 32
B = 16384
V = 1000000

S_TC = 2048
B_SC = B - S_TC

_info = plsc.get_sparse_core_info()
_NC, _NS = _info.num_cores, _info.num_subcores
NW = _NC * _NS
GS = 16
NBLK = B_SC // 128

_mesh = plsc.VectorSubcoreMesh(core_axis_name="c", subcore_axis_name="s")


@functools.partial(
    pl.kernel,
    mesh=_mesh,
    out_type=jax.ShapeDtypeStruct((D, B_SC), jnp.float32),
    scratch_types=[
        pltpu.VMEM((128,), jnp.int32),
        pltpu.VMEM((GS, D, 128), jnp.float32),
        pltpu.VMEM((D, 128), jnp.float32),
        pltpu.SemaphoreType.DMA,
        pltpu.SemaphoreType.DMA,
    ],
    compiler_params=pltpu.CompilerParams(needs_layout_passes=False),
)
def _sc_gather(idx_hbm, embT_hbm, outT_hbm, idx_v, blk_v, ob_v, gsem, osem):
    wid = lax.axis_index("s") * _NC + lax.axis_index("c")
    nblk = (NBLK + NW - 1 - wid) // NW
    iota = lax.iota(jnp.int32, 16)

    def block(t, carry):
        bb = pl.multiple_of((wid + t * NW) * 128, 128)
        pltpu.sync_copy(idx_hbm.at[pl.ds(bb, 128)], idx_v)
        for sub in range(128 // GS):
            rv = idx_v[pl.ds(sub * GS, GS)]
            copies = []
            lanes = []
            for i in range(GS):
                r = rv[i]
                w0 = pl.multiple_of(
                    lax.shift_left(lax.shift_right_logical(r, 7), 7), 128
                )
                lanes.append(r - w0)
                copies.append(
                    pltpu.async_copy(
                        embT_hbm.at[:, pl.ds(w0, 128)], blk_v.at[i], gsem
                    )
                )
            for c in copies:
                c.wait()
            for i in range(GS):
                lane = jnp.full((16,), lanes[i], jnp.int32)
                row = jnp.full((16,), i, jnp.int32)
                col = jnp.full((16,), sub * GS + i, jnp.int32)
                lo = plsc.load_gather(blk_v, [row, iota, lane])
                hi = plsc.load_gather(blk_v, [row, iota + 16, lane])
                plsc.store_scatter(ob_v, [iota, col], lo)
                plsc.store_scatter(ob_v, [iota + 16, col], hi)
        pltpu.async_copy(
            ob_v, outT_hbm.at[:, pl.ds(bb, 128)], osem
        ).wait()
        return carry

    lax.fori_loop(0, nblk, block, 0)


EPG = 8


def _tc_gather_body(wins_ref, lanes_ref, *refs):
    tables = refs[:EPG]
    out_ref = refs[EPG]
    i = pl.program_id(0)
    lane_iota = lax.broadcasted_iota(jnp.int32, (128, 1), 0)
    for k in range(EPG):
        j = lanes_ref[i * EPG + k]
        onehot = (lane_iota == j).astype(jnp.float32)
        blk = tables[k][...]
        col = jax.lax.dot_general(
            blk, onehot, (((1,), (0,)), ((), ())),
            preferred_element_type=jnp.float32,
        )
        out_ref[k, :] = col[:, 0]


def _tc_in_spec(k):
    return pl.BlockSpec((D, 128), lambda i, wins, lanes, k=k: (0, wins[i * EPG + k]))


_tc_gather = pl.pallas_call(
    _tc_gather_body,
    grid_spec=pltpu.PrefetchScalarGridSpec(
        num_scalar_prefetch=2,
        grid=(S_TC // EPG,),
        in_specs=[_tc_in_spec(k) for k in range(EPG)],
        out_specs=pl.BlockSpec((EPG, D), lambda i, wins, lanes: (i, 0)),
    ),
    out_shape=jax.ShapeDtypeStruct((S_TC, D), jnp.float32),
)


def kernel(node_idx, emb):
    idx = node_idx.astype(jnp.int32)
    embT = emb.T
    idx_tc = idx[:S_TC]
    wins = lax.shift_right_logical(idx_tc, 7)
    lanes = jnp.bitwise_and(idx_tc, 127)
    tc_out = _tc_gather(wins, lanes, *([embT] * EPG))
    sc_outT = _sc_gather(idx[S_TC:], embT)
    return jnp.concatenate([tc_out, sc_outT.T], axis=0)


# speedup vs baseline: 2.0981x; 1.2913x over previous
import functools

import jax
import jax.numpy as jnp
from jax import lax
from jax.experimental import pallas as pl
from jax.experimental.pallas import tpu as pltpu
from jax.experimental.pallas import tpu_sc as plsc

D = 32
B = 16384
V = 1000000

S_TC = 1024
B_SC = B - S_TC

_info = plsc.get_sparse_core_info()
_NC, _NS = _info.num_cores, _info.num_subcores
NW = _NC * _NS
GS = 16
NBLK = B_SC // 128

_mesh = plsc.VectorSubcoreMesh(core_axis_name="c", subcore_axis_name="s")


@functools.partial(
    pl.kernel,
    mesh=_mesh,
    out_type=jax.ShapeDtypeStruct((D, B_SC), jnp.float32),
    scratch_types=[
        pltpu.VMEM((128,), jnp.int32),
        pltpu.VMEM((GS, D, 128), jnp.float32),
        pltpu.VMEM((D, 128), jnp.float32),
        pltpu.SemaphoreType.DMA,
        pltpu.SemaphoreType.DMA,
    ],
    compiler_params=pltpu.CompilerParams(needs_layout_passes=False),
)
def _sc_gather(idx_hbm, embT_hbm, outT_hbm, idx_v, blk_v, ob_v, gsem, osem):
    wid = lax.axis_index("s") * _NC + lax.axis_index("c")
    nblk = (NBLK + NW - 1 - wid) // NW
    iota = lax.iota(jnp.int32, 16)

    def block(t, carry):
        bb = pl.multiple_of((wid + t * NW) * 128, 128)
        pltpu.sync_copy(idx_hbm.at[pl.ds(bb, 128)], idx_v)
        for sub in range(128 // GS):
            rv = idx_v[pl.ds(sub * GS, GS)]
            copies = []
            lanes = []
            for i in range(GS):
                r = rv[i]
                w0 = pl.multiple_of(
                    lax.shift_left(lax.shift_right_logical(r, 7), 7), 128
                )
                lanes.append(r - w0)
                copies.append(
                    pltpu.async_copy(
                        embT_hbm.at[:, pl.ds(w0, 128)], blk_v.at[i], gsem
                    )
                )
            for c in copies:
                c.wait()
            for i in range(GS):
                lane = jnp.full((16,), lanes[i], jnp.int32)
                row = jnp.full((16,), i, jnp.int32)
                col = jnp.full((16,), sub * GS + i, jnp.int32)
                lo = plsc.load_gather(blk_v, [row, iota, lane])
                hi = plsc.load_gather(blk_v, [row, iota + 16, lane])
                plsc.store_scatter(ob_v, [iota, col], lo)
                plsc.store_scatter(ob_v, [iota + 16, col], hi)
        pltpu.async_copy(
            ob_v, outT_hbm.at[:, pl.ds(bb, 128)], osem
        ).wait()
        return carry

    lax.fori_loop(0, nblk, block, 0)


EPG = 16


def _tc_gather_body(wins_ref, lanes_ref, *refs):
    tables = refs[:EPG]
    out_ref = refs[EPG]
    i = pl.program_id(0)
    lane_iota = lax.broadcasted_iota(jnp.int32, (D, 128), 1)
    for k in range(EPG):
        j = lanes_ref[i * EPG + k]
        blk = tables[k][...]
        col = jnp.sum(jnp.where(lane_iota == j, blk, 0.0), axis=1)
        out_ref[k, :] = col


def _tc_in_spec(k):
    return pl.BlockSpec((D, 128), lambda i, wins, lanes, k=k: (0, wins[i * EPG + k]))


_tc_gather = pl.pallas_call(
    _tc_gather_body,
    grid_spec=pltpu.PrefetchScalarGridSpec(
        num_scalar_prefetch=2,
        grid=(S_TC // EPG,),
        in_specs=[_tc_in_spec(k) for k in range(EPG)],
        out_specs=pl.BlockSpec((EPG, D), lambda i, wins, lanes: (i, 0)),
    ),
    out_shape=jax.ShapeDtypeStruct((S_TC, D), jnp.float32),
)


def kernel(node_idx, emb):
    idx = node_idx.astype(jnp.int32)
    embT = emb.T
    idx_tc = idx[:S_TC]
    wins = lax.shift_right_logical(idx_tc, 7)
    lanes = jnp.bitwise_and(idx_tc, 127)
    tc_out = _tc_gather(wins, lanes, *([embT] * EPG))
    sc_outT = _sc_gather(idx[S_TC:], embT)
    return jnp.concatenate([tc_out, sc_outT.T], axis=0)


# speedup vs baseline: 2.1964x; 1.0469x over previous
import functools

import jax
import jax.numpy as jnp
from jax import lax
from jax.experimental import pallas as pl
from jax.experimental.pallas import tpu as pltpu
from jax.experimental.pallas import tpu_sc as plsc

D = 32
B = 16384
V = 1000000

_info = plsc.get_sparse_core_info()
_NC, _NS = _info.num_cores, _info.num_subcores
NW = _NC * _NS
BPW = B // NW
GS = 16
NBLK = BPW // 128

_mesh = plsc.VectorSubcoreMesh(core_axis_name="c", subcore_axis_name="s")


@functools.partial(
    pl.kernel,
    mesh=_mesh,
    out_type=jax.ShapeDtypeStruct((D, B), jnp.float32),
    scratch_types=[
        pltpu.VMEM((BPW,), jnp.int32),
        pltpu.VMEM((GS, D, 128), jnp.float32),
        pltpu.VMEM((D, 128), jnp.float32),
        pltpu.SemaphoreType.DMA,
        pltpu.SemaphoreType.DMA,
    ],
    compiler_params=pltpu.CompilerParams(needs_layout_passes=False),
)
def _gather_kernel(idx_hbm, embT_hbm, outT_hbm, idx_v, blk_v, ob_v, gsem, osem):
    wid = lax.axis_index("s") * _NC + lax.axis_index("c")
    base = wid * BPW
    pltpu.sync_copy(idx_hbm.at[pl.ds(base, BPW)], idx_v)
    iota = lax.iota(jnp.int32, 16)

    def block(blki, carry):
        bb = blki * 128
        for sub in range(128 // GS):
            rv = idx_v[pl.ds(bb + sub * GS, GS)]
            copies = []
            lanes = []
            for i in range(GS):
                r = rv[i]
                w0 = pl.multiple_of(
                    lax.shift_left(lax.shift_right_logical(r, 7), 7), 128
                )
                lanes.append(r - w0)
                copies.append(
                    pltpu.async_copy(
                        embT_hbm.at[:, pl.ds(w0, 128)], blk_v.at[i], gsem
                    )
                )
            for c in copies:
                c.wait()
            for i in range(GS):
                lane = jnp.full((16,), lanes[i], jnp.int32)
                row = jnp.full((16,), i, jnp.int32)
                col = jnp.full((16,), sub * GS + i, jnp.int32)
                lo = plsc.load_gather(blk_v, [row, iota, lane])
                hi = plsc.load_gather(blk_v, [row, iota + 16, lane])
                plsc.store_scatter(ob_v, [iota, col], lo)
                plsc.store_scatter(ob_v, [iota + 16, col], hi)
        pltpu.async_copy(
            ob_v, outT_hbm.at[:, pl.ds(base + bb, 128)], osem
        ).wait()
        return carry

    lax.fori_loop(0, NBLK, block, 0)


def kernel(node_idx, emb):
    outT = _gather_kernel(node_idx.astype(jnp.int32), emb.T)
    return outT.T


# speedup vs baseline: 2.3713x; 1.0797x over previous
import functools

import jax
import jax.numpy as jnp
from jax import lax
from jax.experimental import pallas as pl
from jax.experimental.pallas import tpu as pltpu
from jax.experimental.pallas import tpu_sc as plsc

D = 32
B = 16384
V = 1000000

_info = plsc.get_sparse_core_info()
_NC, _NS = _info.num_cores, _info.num_subcores
NW = _NC * _NS
BPW = B // NW
GS = 16
NBLK = BPW // 128

_mesh = plsc.VectorSubcoreMesh(core_axis_name="c", subcore_axis_name="s")


@functools.partial(
    pl.kernel,
    mesh=_mesh,
    out_type=jax.ShapeDtypeStruct((D, B), jnp.float32),
    scratch_types=[
        pltpu.VMEM((BPW,), jnp.int32),
        pltpu.VMEM((GS, D, 128), jnp.float32),
        pltpu.VMEM((D, 128), jnp.float32),
        pltpu.SemaphoreType.DMA,
        pltpu.SemaphoreType.DMA,
    ],
    compiler_params=pltpu.CompilerParams(needs_layout_passes=False),
)
def _gather_kernel(idx_hbm, embT_hbm, outT_hbm, idx_v, blk_v, ob_v, gsem, osem):
    wid = lax.axis_index("s") * _NC + lax.axis_index("c")
    base = wid * BPW
    pltpu.sync_copy(idx_hbm.at[pl.ds(base, BPW)], idx_v)
    iota = lax.iota(jnp.int32, 16)

    def block(blki, carry):
        bb = blki * 128
        for sub in range(128 // GS):
            rv = idx_v[pl.ds(bb + sub * GS, GS)]
            copies = []
            lanes = []
            for i in range(GS):
                r = rv[i]
                w0 = pl.multiple_of(
                    lax.shift_left(lax.shift_right_logical(r, 7), 7), 128
                )
                lanes.append(r - w0)
                copies.append(
                    pltpu.async_copy(
                        embT_hbm.at[:, pl.ds(w0, 128)], blk_v.at[i], gsem
                    )
                )
            for i in range(GS):
                copies[i].wait()
                lane = jnp.full((16,), lanes[i], jnp.int32)
                row = jnp.full((16,), i, jnp.int32)
                col = jnp.full((16,), sub * GS + i, jnp.int32)
                lo = plsc.load_gather(blk_v, [row, iota, lane])
                hi = plsc.load_gather(blk_v, [row, iota + 16, lane])
                plsc.store_scatter(ob_v, [iota, col], lo)
                plsc.store_scatter(ob_v, [iota + 16, col], hi)
        pltpu.async_copy(
            ob_v, outT_hbm.at[:, pl.ds(base + bb, 128)], osem
        ).wait()
        return carry

    lax.fori_loop(0, NBLK, block, 0)


def kernel(node_idx, emb):
    outT = _gather_kernel(node_idx.astype(jnp.int32), emb.T)
    return outT.T


# speedup vs baseline: 2.8157x; 1.1874x over previous
import functools

import jax
import jax.numpy as jnp
from jax import lax
from jax.experimental import pallas as pl
from jax.experimental.pallas import tpu as pltpu
from jax.experimental.pallas import tpu_sc as plsc

D = 32
B = 16384
V = 1000000

_info = plsc.get_sparse_core_info()
_NC, _NS = _info.num_cores, _info.num_subcores
NW = _NC * _NS
BPW = B // NW
GS = 16
NBLK = BPW // 128

_mesh = plsc.VectorSubcoreMesh(core_axis_name="c", subcore_axis_name="s")


@functools.partial(
    pl.kernel,
    mesh=_mesh,
    out_type=jax.ShapeDtypeStruct((D, B), jnp.float32),
    scratch_types=[
        pltpu.VMEM((BPW,), jnp.int32),
        pltpu.VMEM((GS, D, 128), jnp.float32),
        pltpu.VMEM((D, 128), jnp.float32),
        pltpu.SemaphoreType.DMA,
        pltpu.SemaphoreType.DMA,
    ],
    compiler_params=pltpu.CompilerParams(needs_layout_passes=False),
)
def _gather_kernel(idx_hbm, embT_hbm, outT_hbm, idx_v, blk_v, ob_v, gsem, osem):
    wid = lax.axis_index("s") * _NC + lax.axis_index("c")
    base = wid * BPW
    pltpu.sync_copy(idx_hbm.at[pl.ds(base, BPW)], idx_v)
    iota = lax.iota(jnp.int32, 16)

    def burst_args(bb, sub):
        rv = idx_v[pl.ds(bb + sub * GS, GS)]
        lanes, w0s = [], []
        for i in range(GS):
            r = rv[i]
            w0 = pl.multiple_of(
                lax.shift_left(lax.shift_right_logical(r, 7), 7), 128
            )
            w0s.append(w0)
            lanes.append(r - w0)
        return lanes, w0s

    def issue(w0, slot):
        return pltpu.async_copy(
            embT_hbm.at[:, pl.ds(w0, 128)], blk_v.at[slot], gsem
        )

    def block(blki, carry):
        bb = blki * 128
        lanes_cur, w0s_cur = burst_args(bb, 0)
        copies_cur = [issue(w0s_cur[i], i) for i in range(GS)]
        for sub in range(128 // GS):
            nxt = sub + 1
            if nxt < 128 // GS:
                lanes_nxt, w0s_nxt = burst_args(bb, nxt)
            copies_nxt = []
            for i in range(GS):
                copies_cur[i].wait()
                lane = jnp.full((16,), lanes_cur[i], jnp.int32)
                row = jnp.full((16,), i, jnp.int32)
                col = jnp.full((16,), sub * GS + i, jnp.int32)
                lo = plsc.load_gather(blk_v, [row, iota, lane])
                hi = plsc.load_gather(blk_v, [row, iota + 16, lane])
                plsc.store_scatter(ob_v, [iota, col], lo)
                plsc.store_scatter(ob_v, [iota + 16, col], hi)
                if nxt < 128 // GS:
                    copies_nxt.append(issue(w0s_nxt[i], i))
            if nxt < 128 // GS:
                lanes_cur, copies_cur = lanes_nxt, copies_nxt
        pltpu.async_copy(
            ob_v, outT_hbm.at[:, pl.ds(base + bb, 128)], osem
        ).wait()
        return carry

    lax.fori_loop(0, NBLK, block, 0)


def kernel(node_idx, emb):
    outT = _gather_kernel(node_idx.astype(jnp.int32), emb.T)
    return outT.T
